# baseline (speedup 1.0000x reference)
import functools

import jax
import jax.numpy as jnp
from jax import lax
from jax.experimental import pallas as pl
from jax.experimental.pallas import tpu as pltpu
from jax.experimental.pallas import tpu_sc as plsc

_NC = 2
_NS = 16
_NW = _NC * _NS
_L = 16
_NCHK = 8


def kernel(x):
    n, d = x.shape
    assert n % _NW == 0 and d >= _L
    rows = n // _NW
    csizes = [rows // 4, rows // 4, rows // 8, rows // 8,
              rows // 16, rows // 16, rows // 16, rows // 16]
    csizes[0] += rows - sum(csizes)
    cstarts = [sum(csizes[:i]) for i in range(_NCHK)]
    assert all(c % 8 == 0 for c in csizes)
    mesh = plsc.VectorSubcoreMesh(core_axis_name="c", subcore_axis_name="s")

    dn = lax.GatherDimensionNumbers(
        offset_dims=(), collapsed_slice_dims=(0,), start_index_map=(0,))

    def perm(v, idx):
        return lax.gather(v, idx, dn, slice_sizes=(1,),
                          mode=lax.GatherScatterMode.PROMISE_IN_BOUNDS)

    @functools.partial(
        pl.kernel,
        out_type=jax.ShapeDtypeStruct((n * d,), x.dtype),
        mesh=mesh,
        scratch_types=[pltpu.VMEM((rows * d,), jnp.float32)]
        + [pltpu.SemaphoreType.DMA] * (2 * _NCHK),
    )
    def run(x_hbm, out_hbm, slab, *sems):
        wid = lax.axis_index("s") * _NC + lax.axis_index("c")
        base = wid * rows

        in_cps = []
        for k in range(_NCHK):
            hbm_sl = pl.ds((base + cstarts[k]) * d, csizes[k] * d)
            loc_sl = pl.ds(cstarts[k] * d, csizes[k] * d)
            in_cps.append(
                pltpu.async_copy(x_hbm.at[hbm_sl], slab.at[loc_sl], sems[k]))

        lane = lax.iota(jnp.int32, _L)
        m13 = lane >= 13
        m14 = lane == 14
        zero = jnp.zeros((_L,), jnp.float32)
        one = jnp.ones((_L,), jnp.float32)
        izero = jnp.zeros((_L,), jnp.int32)
        ione = izero + 1
        e13 = jnp.where(lane == 13, ione, izero)
        e14 = jnp.where(m14, ione, izero)
        e15 = jnp.where(lane == 15, ione, izero)
        i_num = (lane - e14)[:, None]
        i_d2 = (lane - e13 - e15)[:, None]
        i_d3 = (lane + e14)[:, None]

        out_cps = []
        for k in range(_NCHK):
            in_cps[k].wait()

            def step(r, carry, k=k):
                for u in range(8):
                    off = (cstarts[k] + r * 8 + u) * d + (d - _L)
                    v = slab[pl.ds(off, _L)]
                    num = perm(v, i_num)
                    d2 = perm(v, i_d2)
                    d3 = perm(v, i_d3)
                    den = num + d2 + jnp.where(m14, d3, zero)
                    slab[pl.ds(off, _L)] = (
                        jnp.where(m13, num, v) / jnp.where(m13, den, one)
                    )
                return carry

            lax.fori_loop(0, csizes[k] // 8, step, 0)
            hbm_sl = pl.ds((base + cstarts[k]) * d, csizes[k] * d)
            loc_sl = pl.ds(cstarts[k] * d, csizes[k] * d)
            out_cps.append(
                pltpu.async_copy(slab.at[loc_sl], out_hbm.at[hbm_sl],
                                 sems[_NCHK + k]))
        for cp in out_cps:
            cp.wait()

    return run(x.reshape(-1)).reshape(n, d)


# speedup vs baseline: 1.0054x; 1.0054x over previous
import functools

import jax
import jax.numpy as jnp
from jax import lax
from jax.experimental import pallas as pl
from jax.experimental.pallas import tpu as pltpu
from jax.experimental.pallas import tpu_sc as plsc

_NC = 2
_NS = 16
_NW = _NC * _NS
_L = 16
_NCHK = 8


def kernel(x):
    n, d = x.shape
    assert n % (_NW * _NCHK) == 0 and d >= _L
    rows = n // _NW
    crows = rows // _NCHK
    mesh = plsc.VectorSubcoreMesh(core_axis_name="c", subcore_axis_name="s")

    dn = lax.GatherDimensionNumbers(
        offset_dims=(), collapsed_slice_dims=(0,), start_index_map=(0,))

    def perm(v, idx):
        return lax.gather(v, idx, dn, slice_sizes=(1,),
                          mode=lax.GatherScatterMode.PROMISE_IN_BOUNDS)

    @functools.partial(
        pl.kernel,
        out_type=jax.ShapeDtypeStruct((n * d,), x.dtype),
        mesh=mesh,
        scratch_types=[pltpu.VMEM((rows * d,), jnp.float32)]
        + [pltpu.SemaphoreType.DMA] * (2 * _NCHK),
    )
    def run(x_hbm, out_hbm, slab, *sems):
        wid = lax.axis_index("s") * _NC + lax.axis_index("c")
        base = wid * rows

        in_cps = []
        for k in range(_NCHK):
            hbm_sl = pl.ds((base + k * crows) * d, crows * d)
            loc_sl = pl.ds(k * crows * d, crows * d)
            in_cps.append(
                pltpu.async_copy(x_hbm.at[hbm_sl], slab.at[loc_sl], sems[k]))

        lane = lax.iota(jnp.int32, _L)
        m13 = lane >= 13
        m14 = lane == 14
        zero = jnp.zeros((_L,), jnp.float32)
        one = jnp.ones((_L,), jnp.float32)
        izero = jnp.zeros((_L,), jnp.int32)
        ione = izero + 1
        e13 = jnp.where(lane == 13, ione, izero)
        e14 = jnp.where(m14, ione, izero)
        e15 = jnp.where(lane == 15, ione, izero)
        i_num = (lane - e14)[:, None]
        i_d2 = (lane - e13 - e15)[:, None]
        i_d3 = (lane + e14)[:, None]

        out_cps = []
        for k in range(_NCHK):
            in_cps[k].wait()

            def step(r, carry, k=k):
                for u in range(4):
                    off = (k * crows + r * 4 + u) * d + (d - _L)
                    v = slab[pl.ds(off, _L)]
                    num = perm(v, i_num)
                    d2 = perm(v, i_d2)
                    d3 = perm(v, i_d3)
                    den = num + d2 + jnp.where(m14, d3, zero)
                    slab[pl.ds(off, _L)] = (
                        jnp.where(m13, num, v) / jnp.where(m13, den, one)
                    )
                return carry

            lax.fori_loop(0, crows // 4, step, 0)
            hbm_sl = pl.ds((base + k * crows) * d, crows * d)
            loc_sl = pl.ds(k * crows * d, crows * d)
            out_cps.append(
                pltpu.async_copy(slab.at[loc_sl], out_hbm.at[hbm_sl],
                                 sems[_NCHK + k]))
        for cp in out_cps:
            cp.wait()

    return run(x.reshape(-1)).reshape(n, d)


# speedup vs baseline: 1.0257x; 1.0202x over previous
import functools

import jax
import jax.numpy as jnp
from jax import lax
from jax.experimental import pallas as pl
from jax.experimental.pallas import tpu as pltpu
from jax.experimental.pallas import tpu_sc as plsc

_NC = 2
_NS = 16
_NW = _NC * _NS
_L = 16
_NCHK = 8


def kernel(x):
    n, d = x.shape
    assert n % (_NW * _NCHK) == 0 and d >= _L
    rows = n // _NW
    crows = rows // _NCHK
    mesh = plsc.VectorSubcoreMesh(core_axis_name="c", subcore_axis_name="s")

    dn = lax.GatherDimensionNumbers(
        offset_dims=(), collapsed_slice_dims=(0,), start_index_map=(0,))

    def perm(v, idx):
        return lax.gather(v, idx, dn, slice_sizes=(1,),
                          mode=lax.GatherScatterMode.PROMISE_IN_BOUNDS)

    @functools.partial(
        pl.kernel,
        out_type=jax.ShapeDtypeStruct((n * d,), x.dtype),
        mesh=mesh,
        scratch_types=[pltpu.VMEM((rows * d,), jnp.float32)]
        + [pltpu.SemaphoreType.DMA] * (2 * _NCHK),
    )
    def run(x_hbm, out_hbm, slab, *sems):
        wid = lax.axis_index("s") * _NC + lax.axis_index("c")
        base = wid * rows

        in_cps = []
        for k in range(_NCHK):
            hbm_sl = pl.ds((base + k * crows) * d, crows * d)
            loc_sl = pl.ds(k * crows * d, crows * d)
            in_cps.append(
                pltpu.async_copy(x_hbm.at[hbm_sl], slab.at[loc_sl], sems[k]))

        lane = lax.iota(jnp.int32, _L)
        m13 = lane >= 13
        m14 = lane == 14
        zero = jnp.zeros((_L,), jnp.float32)
        one = jnp.ones((_L,), jnp.float32)
        izero = jnp.zeros((_L,), jnp.int32)
        ione = izero + 1
        e13 = jnp.where(lane == 13, ione, izero)
        e14 = jnp.where(m14, ione, izero)
        e15 = jnp.where(lane == 15, ione, izero)
        i_num = (lane - e14)[:, None]
        i_d2 = (lane - e13 - e15)[:, None]
        i_d3 = (lane + e14)[:, None]

        out_cps = []
        for k in range(_NCHK):
            in_cps[k].wait()

            @plsc.parallel_loop(0, crows, unroll=4)
            def step(r, k=k):
                off = (k * crows + r) * d + (d - _L)
                v = slab[pl.ds(off, _L)]
                num = perm(v, i_num)
                d2 = perm(v, i_d2)
                d3 = perm(v, i_d3)
                den = num + d2 + jnp.where(m14, d3, zero)
                slab[pl.ds(off, _L)] = (
                    jnp.where(m13, num, v) / jnp.where(m13, den, one)
                )
            hbm_sl = pl.ds((base + k * crows) * d, crows * d)
            loc_sl = pl.ds(k * crows * d, crows * d)
            out_cps.append(
                pltpu.async_copy(slab.at[loc_sl], out_hbm.at[hbm_sl],
                                 sems[_NCHK + k]))
        for cp in out_cps:
            cp.wait()

    return run(x.reshape(-1)).reshape(n, d)
